# baseline (speedup 1.0000x reference)
import jax
import jax.numpy as jnp
from jax import lax
from jax.experimental import pallas as pl
from jax.experimental.pallas import tpu as pltpu
from jax.experimental.pallas import tpu_sc as plsc

M, D, B = 100000, 128, 16384
NC, NS, L = 2, 16, 16
NW = NC * NS
RPW = B // NW
CH = 16
NCH = RPW // CH
NBUF = 4
NVREG = B // L
SLICE = NVREG // NCH


def _body(mem_hbm, widx_hbm, wval_hbm, ridx_hbm, out_hbm,
          table_v, widx_v, ridx_v, cpos_v, csrc_v, rbuf, vbuf,
          sg0, sg1, sg2, sg3, so0, so1, so2, so3, se_g, se_s):
  wid = lax.axis_index("s") * NC + lax.axis_index("c")
  rbase = wid * RPW
  iota = lax.iota(jnp.int32, L)
  sgs = (sg0, sg1, sg2, sg3)
  sos = (so0, so1, so2, so3)

  pltpu.sync_copy(ridx_hbm.at[pl.ds(rbase, RPW)], ridx_v)
  g_desc = {}
  o_desc = {}
  for k in range(NBUF):
    g_desc[k] = pltpu.async_copy(
        mem_hbm.at[ridx_v.at[pl.ds(k * CH, CH)]], rbuf.at[k], sgs[k])
  pltpu.sync_copy(widx_hbm, widx_v)

  def scan_body(i, _):
    idxv = widx_v[pl.ds(i * L, L)]
    _, lastm = plsc.scan_count(idxv)
    jv = 1 + i * L + iota
    plsc.store_scatter(table_v, [idxv], jv, mask=lastm)
    return 0

  for k in range(NCH):
    slot = k % NBUF
    g_desc[k].wait()
    o_desc[k] = pltpu.async_copy(
        rbuf.at[slot], out_hbm.at[pl.ds(rbase + k * CH, CH)], sos[slot])
    lax.fori_loop(k * SLICE, (k + 1) * SLICE, scan_body, 0, unroll=8)
    nk = k + NBUF
    if nk < NCH:
      o_desc[k].wait()
      g_desc[nk] = pltpu.async_copy(
          mem_hbm.at[ridx_v.at[pl.ds(nk * CH, CH)]], rbuf.at[slot], sgs[slot])
  for k in range(NCH - NBUF, NCH):
    o_desc[k].wait()

  def d_body(i, nv):
    rv = ridx_v[pl.ds(i * L, L)]
    tv = plsc.load_gather(table_v, [rv])
    tc = jnp.clip(tv - 1, 0, B - 1)
    wi = plsc.load_gather(widx_v, [tc])
    hit = (tv > 0) & (tv <= B) & (wi == rv)
    pos = rbase + i * L + iota
    plsc.store_compressed(cpos_v.at[pl.ds(nv, L)], pos, mask=hit)
    plsc.store_compressed(csrc_v.at[pl.ds(nv, L)], tc, mask=hit)
    return nv + jnp.sum(hit.astype(jnp.int32))

  nv = lax.fori_loop(0, RPW // L, d_body, 0)

  safe = jnp.maximum(nv - 1, 0)
  fill_p = jnp.full((L,), cpos_v[pl.ds(safe, L)][0], jnp.int32)
  fill_s = jnp.full((L,), csrc_v[pl.ds(safe, L)][0], jnp.int32)
  cpos_v[pl.ds(nv, L)] = fill_p
  csrc_v[pl.ds(nv, L)] = fill_s

  def e_body(k, _):
    sv = csrc_v[pl.ds(k * L, L)]
    pv = cpos_v[pl.ds(k * L, L)]
    pltpu.async_copy(wval_hbm.at[sv], vbuf, se_g).wait()
    pltpu.async_copy(vbuf, out_hbm.at[pv], se_s).wait()
    return 0

  lax.fori_loop(0, (nv + L - 1) // L, e_body, 0)


_mrw = pl.kernel(
    _body,
    out_type=jax.ShapeDtypeStruct((B, D), jnp.float32),
    mesh=plsc.VectorSubcoreMesh(core_axis_name="c", subcore_axis_name="s",
                                num_cores=NC, num_subcores=NS),
    compiler_params=pltpu.CompilerParams(needs_layout_passes=False),
    scratch_types=[
        pltpu.VMEM((M,), jnp.int32),
        pltpu.VMEM((B,), jnp.int32),
        pltpu.VMEM((RPW,), jnp.int32),
        pltpu.VMEM((RPW + L,), jnp.int32),
        pltpu.VMEM((RPW + L,), jnp.int32),
        pltpu.VMEM((NBUF, CH, D), jnp.float32),
        pltpu.VMEM((L, D), jnp.float32),
        pltpu.SemaphoreType.DMA,
        pltpu.SemaphoreType.DMA,
        pltpu.SemaphoreType.DMA,
        pltpu.SemaphoreType.DMA,
        pltpu.SemaphoreType.DMA,
        pltpu.SemaphoreType.DMA,
        pltpu.SemaphoreType.DMA,
        pltpu.SemaphoreType.DMA,
        pltpu.SemaphoreType.DMA,
        pltpu.SemaphoreType.DMA,
    ],
)


def kernel(mem, write_idx, write_val, read_idx):
  return _mrw(mem, write_idx, write_val, read_idx)


# speedup vs baseline: 1.1335x; 1.1335x over previous
import jax
import jax.numpy as jnp
from jax import lax
from jax.experimental import pallas as pl
from jax.experimental.pallas import tpu as pltpu
from jax.experimental.pallas import tpu_sc as plsc

M, D, B = 100000, 128, 16384
NC, NS, L = 2, 16, 16
NW = NC * NS
RPW = B // NW
CH = 16
NCH = RPW // CH
NBUF = 4
NVREG = B // L
SLICE = NVREG // NCH
P2S = 128


def _body(mem_hbm, widx_hbm, wval_hbm, ridx_hbm, out_hbm,
          table_v, widx_v, ridx_v, cpos_v, csrc_v, rbuf, vbuf,
          sg0, sg1, sg2, sg3, so0, so1, so2, so3, se_g, se_g2, se_s, se_s2):
  wid = lax.axis_index("s") * NC + lax.axis_index("c")
  rbase = wid * RPW
  iota = lax.iota(jnp.int32, L)
  sgs = (sg0, sg1, sg2, sg3)
  sos = (so0, so1, so2, so3)

  pltpu.sync_copy(ridx_hbm.at[pl.ds(rbase, RPW)], ridx_v)
  g_desc = {}
  o_desc = {}
  for k in range(NBUF):
    g_desc[k] = pltpu.async_copy(
        mem_hbm.at[ridx_v.at[pl.ds(k * CH, CH)]], rbuf.at[k], sgs[k])
  pltpu.sync_copy(widx_hbm, widx_v)

  def scat_body(i, _):
    idxv = widx_v[pl.ds(i * L, L)]
    jv = 1 + i * L + iota
    plsc.store_scatter(table_v, [idxv], jv)
    return 0

  for k in range(NCH):
    slot = k % NBUF
    g_desc[k].wait()
    o_desc[k] = pltpu.async_copy(
        rbuf.at[slot], out_hbm.at[pl.ds(rbase + k * CH, CH)], sos[slot])
    lax.fori_loop(k * SLICE, (k + 1) * SLICE, scat_body, 0, unroll=8)
    nk = k + NBUF
    if nk < NCH:
      o_desc[k].wait()
      g_desc[nk] = pltpu.async_copy(
          mem_hbm.at[ridx_v.at[pl.ds(nk * CH, CH)]], rbuf.at[slot], sgs[slot])
  for k in range(NCH - NBUF, NCH):
    o_desc[k].wait()

  def chk_body(i, acc):
    idxv = widx_v[pl.ds(i * L, L)]
    jv = 1 + i * L + iota
    g = plsc.load_gather(table_v, [idxv])
    return acc | (g < jv)

  def fix_body(i, _):
    idxv = widx_v[pl.ds(i * L, L)]
    _, lastm = plsc.scan_count(idxv)
    jv = 1 + i * L + iota
    g = plsc.load_gather(table_v, [idxv])
    mine = (g > i * L) & (g <= i * L + L)
    plsc.store_scatter(table_v, [idxv], jv, mask=lastm & mine & (g < jv))
    return 0

  for s in range(NVREG // P2S):
    acc = lax.fori_loop(s * P2S, (s + 1) * P2S, chk_body, iota < 0, unroll=8)

    @pl.when(jnp.any(acc))
    def _(s=s):
      lax.fori_loop(s * P2S, (s + 1) * P2S, fix_body, 0)

  def d_body(i, nv):
    rv = ridx_v[pl.ds(i * L, L)]
    tv = plsc.load_gather(table_v, [rv])
    tc = jnp.clip(tv - 1, 0, B - 1)
    wi = plsc.load_gather(widx_v, [tc])
    hit = (tv > 0) & (tv <= B) & (wi == rv)
    pos = rbase + i * L + iota
    plsc.store_compressed(cpos_v.at[pl.ds(nv, L)], pos, mask=hit)
    plsc.store_compressed(csrc_v.at[pl.ds(nv, L)], tc, mask=hit)
    return nv + jnp.sum(hit.astype(jnp.int32))

  nv = lax.fori_loop(0, RPW // L, d_body, 0)

  safe = jnp.maximum(nv - 1, 0)
  fill_p = jnp.full((L,), cpos_v[pl.ds(safe, L)][0], jnp.int32)
  fill_s = jnp.full((L,), csrc_v[pl.ds(safe, L)][0], jnp.int32)
  cpos_v[pl.ds(nv, L)] = fill_p
  csrc_v[pl.ds(nv, L)] = fill_s
  cpos_v[pl.ds(nv + L, L)] = fill_p
  csrc_v[pl.ds(nv + L, L)] = fill_s

  nE = (nv + L - 1) // L

  def e_pair(m, _):
    k0 = 2 * m
    k1 = k0 + 1
    sv0 = csrc_v[pl.ds(k0 * L, L)]
    pv0 = cpos_v[pl.ds(k0 * L, L)]
    g0 = pltpu.async_copy(wval_hbm.at[sv0], vbuf.at[0], se_g)
    run1 = k1 < nE

    @pl.when(run1)
    def _():
      sv1 = csrc_v[pl.ds(k1 * L, L)]
      pltpu.async_copy(wval_hbm.at[sv1], vbuf.at[1], se_g2)

    g0.wait()
    s0 = pltpu.async_copy(vbuf.at[0], out_hbm.at[pv0], se_s)

    @pl.when(run1)
    def _():
      sv1 = csrc_v[pl.ds(k1 * L, L)]
      pv1 = cpos_v[pl.ds(k1 * L, L)]
      pltpu.make_async_copy(wval_hbm.at[sv1], vbuf.at[1], se_g2).wait()
      pltpu.async_copy(vbuf.at[1], out_hbm.at[pv1], se_s2).wait()

    s0.wait()
    return 0

  lax.fori_loop(0, (nE + 1) // 2, e_pair, 0)


_mrw = pl.kernel(
    _body,
    out_type=jax.ShapeDtypeStruct((B, D), jnp.float32),
    mesh=plsc.VectorSubcoreMesh(core_axis_name="c", subcore_axis_name="s",
                                num_cores=NC, num_subcores=NS),
    compiler_params=pltpu.CompilerParams(needs_layout_passes=False),
    scratch_types=[
        pltpu.VMEM((M,), jnp.int32),
        pltpu.VMEM((B,), jnp.int32),
        pltpu.VMEM((RPW,), jnp.int32),
        pltpu.VMEM((RPW + 2 * L,), jnp.int32),
        pltpu.VMEM((RPW + 2 * L,), jnp.int32),
        pltpu.VMEM((NBUF, CH, D), jnp.float32),
        pltpu.VMEM((2, L, D), jnp.float32),
        pltpu.SemaphoreType.DMA,
        pltpu.SemaphoreType.DMA,
        pltpu.SemaphoreType.DMA,
        pltpu.SemaphoreType.DMA,
        pltpu.SemaphoreType.DMA,
        pltpu.SemaphoreType.DMA,
        pltpu.SemaphoreType.DMA,
        pltpu.SemaphoreType.DMA,
        pltpu.SemaphoreType.DMA,
        pltpu.SemaphoreType.DMA,
        pltpu.SemaphoreType.DMA,
        pltpu.SemaphoreType.DMA,
    ],
)


def kernel(mem, write_idx, write_val, read_idx):
  return _mrw(mem, write_idx, write_val, read_idx)
